# baseline (speedup 1.0000x reference)
import functools

import jax
import jax.numpy as jnp
from jax import lax
from jax.experimental import pallas as pl
from jax.experimental.pallas import tpu as pltpu
from jax.experimental.pallas import tpu_sc as plsc

N = 50000
K = 32
D = 128
B = 16384

_info = plsc.get_sparse_core_info()
_NC, _NS, _LANES = _info.num_cores, _info.num_subcores, _info.num_lanes
_NW = _NC * _NS
_CB = B // _NW
_RING = 4
_QB = 128
_NQ = _CB // _QB
_IDXW = 128
_NJ = _CB // _IDXW
_NPR = D // K


def _sc_body(nid_hbm, table_hbm, ids8_hbm, w8_hbm, xs_out, ws_out,
             nid_v, nidq_v, idraw_v, wraw_v, idc_v, wc_v, self_v,
             rows0, rows1, rows2, rows3, out_v,
             sem_raw, sem_self, sem0, sem1, sem2, sem3):
    wid = lax.axis_index("s") * _NC + lax.axis_index("c")
    base = wid * _CB

    pltpu.sync_copy(nid_hbm.at[pl.ds(wid * _NJ, _NJ)], nid_v)

    bufs = (rows0, rows1, rows2, rows3)
    sems = (sem0, sem1, sem2, sem3)
    iota16 = lax.iota(jnp.int32, 16)

    def _rows_copy(b, buf, sem):
        return pltpu.make_async_copy(table_hbm.at[idc_v.at[b]], buf, sem)

    def _one_b(b, buf, row_local):
        w0 = wc_v[b, pl.ds(0, 16)]
        w1 = wc_v[b, pl.ds(16, 16)]
        accs = [jnp.zeros((16,), jnp.float32) for _ in range(D // 16)]
        for k in range(K):
            wsc = (w0 if k < 16 else w1)[k % 16]
            for fb in range(D // 16):
                accs[fb] = accs[fb] + buf[k, pl.ds(fb * 16, 16)] * wsc
        for fb in range(D // 16):
            out_v[row_local, pl.ds(fb * 16, 16)] = accs[fb]

    def _quarter(q, carry):
        def _mkq(g, c):
            nidq_v[pl.ds(g * 16, 16)] = (
                lax.shift_right_logical(nid_v[q, pl.ds(g * 16, 16)], 2))
            return c
        lax.fori_loop(0, _QB // 16, _mkq, 0)

        cp1 = pltpu.make_async_copy(ids8_hbm.at[nidq_v], idraw_v, sem_raw)
        cp2 = pltpu.make_async_copy(w8_hbm.at[nidq_v], wraw_v, sem_raw)
        cp1.start()
        cp2.start()
        cp1.wait()
        cp2.wait()

        def _ext(g, c):
            rowv = g * 16 + iota16
            nvec = nid_v[q, pl.ds(g * 16, 16)]
            colbase = (nvec & 3) * 32
            for k in range(K):
                kvec = jnp.full((16,), k, jnp.int32)
                v = plsc.load_gather(idraw_v, [rowv, colbase + k])
                plsc.store_scatter(idc_v, [rowv, kvec], v)
                w = plsc.load_gather(wraw_v, [rowv, colbase + k])
                plsc.store_scatter(wc_v, [rowv, kvec], w)
            return c
        lax.fori_loop(0, _QB // 16, _ext, 0)

        for p in range(_RING):
            _rows_copy(p, bufs[p], sems[p]).start()

        def _g(g, c):
            for p in range(_RING):
                lb = g * _RING + p
                _rows_copy(lb, bufs[p], sems[p]).wait()
                _one_b(lb, bufs[p], lb)
                nlb = jnp.minimum(lb + _RING, _QB - 1)
                _rows_copy(nlb, bufs[p], sems[p]).start()
            return c
        lax.fori_loop(0, _QB // _RING, _g, 0)

        for p in range(_RING):
            _rows_copy(_QB - 1, bufs[p], sems[p]).wait()

        pltpu.sync_copy(out_v, ws_out.at[pl.ds(base + q * _QB, _QB)])
        return carry

    lax.fori_loop(0, _NQ, _quarter, 0)

    for j in range(_NJ):
        cp = pltpu.make_async_copy(table_hbm.at[nid_v.at[j]], self_v, sem_self)
        cp.start()
        cp.wait()
        pltpu.sync_copy(self_v, xs_out.at[pl.ds(base + j * _IDXW, _IDXW)])


_sc_gather = functools.partial(
    pl.kernel,
    out_type=(jax.ShapeDtypeStruct((B, D), jnp.float32),
              jax.ShapeDtypeStruct((B, D), jnp.float32)),
    mesh=plsc.VectorSubcoreMesh(core_axis_name="c", subcore_axis_name="s"),
    scratch_types=[
        pltpu.VMEM((_NJ, _IDXW), jnp.int32),
        pltpu.VMEM((_QB,), jnp.int32),
        pltpu.VMEM((_QB, _IDXW), jnp.int32),
        pltpu.VMEM((_QB, _IDXW), jnp.float32),
        pltpu.VMEM((_QB, K), jnp.int32),
        pltpu.VMEM((_QB, K), jnp.float32),
        pltpu.VMEM((_IDXW, D), jnp.float32),
        pltpu.VMEM((K, D), jnp.float32),
        pltpu.VMEM((K, D), jnp.float32),
        pltpu.VMEM((K, D), jnp.float32),
        pltpu.VMEM((K, D), jnp.float32),
        pltpu.VMEM((_QB, D), jnp.float32),
    ] + [pltpu.SemaphoreType.DMA] * 6,
    compiler_params=pltpu.CompilerParams(use_tc_tiling_on_sc=False,
                                         needs_layout_passes=False),
)(_sc_body)


def _tc_body(w_ref, x_ref, wn_ref, ws_ref, wc_ref, b1_ref, bc_ref, o_ref):
    h = jnp.dot(w_ref[...], wn_ref[...], preferred_element_type=jnp.float32)
    h = h + jnp.dot(x_ref[...], ws_ref[...], preferred_element_type=jnp.float32)
    h = h + b1_ref[...]
    o = jnp.dot(h, wc_ref[...], preferred_element_type=jnp.float32) + bc_ref[...]
    o_ref[...] = jnp.maximum(o, 0.0)


_TC_BLK = 2048


def _tc_linear(wsum, xs, wnT, wsT, wcT, b1, bc):
    return pl.pallas_call(
        _tc_body,
        grid=(B // _TC_BLK,),
        in_specs=[
            pl.BlockSpec((_TC_BLK, D), lambda i: (i, 0)),
            pl.BlockSpec((_TC_BLK, D), lambda i: (i, 0)),
            pl.BlockSpec((D, D), lambda i: (0, 0)),
            pl.BlockSpec((D, D), lambda i: (0, 0)),
            pl.BlockSpec((D, D), lambda i: (0, 0)),
            pl.BlockSpec((1, D), lambda i: (0, 0)),
            pl.BlockSpec((1, D), lambda i: (0, 0)),
        ],
        out_specs=pl.BlockSpec((_TC_BLK, D), lambda i: (i, 0)),
        out_shape=jax.ShapeDtypeStruct((B, D), jnp.float32),
    )(wsum, xs, wnT, wsT, wcT, b1, bc)


def kernel(node_ids, global_emb_table, offline_nbr_ids, offline_nbr_weights,
           Wn, bn, Ws, bs, Wc, bc):
    nid2 = node_ids.astype(jnp.int32).reshape(B // _IDXW, _IDXW)
    ids8 = offline_nbr_ids.astype(jnp.int32).reshape(N * K // _IDXW, _IDXW)
    w8 = offline_nbr_weights.reshape(N * K // _IDXW, _IDXW)
    xs, ws = _sc_gather(nid2, global_emb_table, ids8, w8)
    l = Wn.shape[0] - 1
    b1 = (bn[l] + bs[l]).reshape(1, D)
    return _tc_linear(ws, xs, Wn[l].T, Ws[l].T, Wc[l].T, b1, bc[l].reshape(1, D))


# speedup vs baseline: 1.1935x; 1.1935x over previous
import functools

import jax
import jax.numpy as jnp
from jax import lax
from jax.experimental import pallas as pl
from jax.experimental.pallas import tpu as pltpu
from jax.experimental.pallas import tpu_sc as plsc

N = 50000
K = 32
D = 128
B = 16384

_info = plsc.get_sparse_core_info()
_NC, _NS, _LANES = _info.num_cores, _info.num_subcores, _info.num_lanes
_NW = _NC * _NS
_CB = B // _NW
_RING = 4
_OUTROWS = 128
_IDXW = 128
_NJ = _CB // _IDXW


def _sc_body(nid_hbm, table_hbm, nbrids_hbm, nbrw_hbm, xs_out, ws_out,
             nid_v, nbrids_v, nbrw_v, self_v, rows0, rows1, rows2, rows3,
             out_v, sem_ids, sem_w, sem_self, sem0, sem1, sem2, sem3):
    wid = lax.axis_index("s") * _NC + lax.axis_index("c")
    base = wid * _CB

    pltpu.sync_copy(nid_hbm.at[pl.ds(wid * _NJ, _NJ)], nid_v)

    cps = [pltpu.make_async_copy(
        nbrids_hbm.at[nid_v.at[j]], nbrids_v.at[pl.ds(j * _IDXW, _IDXW)],
        sem_ids) for j in range(_NJ)]
    cps += [pltpu.make_async_copy(
        nbrw_hbm.at[nid_v.at[j]], nbrw_v.at[pl.ds(j * _IDXW, _IDXW)],
        sem_w) for j in range(_NJ)]
    for cp in cps:
        cp.start()
    for cp in cps:
        cp.wait()

    bufs = (rows0, rows1, rows2, rows3)
    sems = (sem0, sem1, sem2, sem3)

    def _rows_copy(b, buf, sem):
        return pltpu.make_async_copy(table_hbm.at[nbrids_v.at[b]], buf, sem)

    for p in range(_RING):
        _rows_copy(p, bufs[p], sems[p]).start()

    def _one_b(b, buf, row_local):
        w0 = nbrw_v[b, pl.ds(0, 16)]
        w1 = nbrw_v[b, pl.ds(16, 16)]
        accs = [jnp.zeros((16,), jnp.float32) for _ in range(D // 16)]
        for k in range(K):
            wsc = (w0 if k < 16 else w1)[k % 16]
            for fb in range(D // 16):
                accs[fb] = accs[fb] + buf[k, pl.ds(fb * 16, 16)] * wsc
        for fb in range(D // 16):
            out_v[row_local, pl.ds(fb * 16, 16)] = accs[fb]

    for half in range(_CB // _OUTROWS):
        def _g(g, carry):
            for p in range(_RING):
                b = half * _OUTROWS + g * _RING + p
                _rows_copy(b, bufs[p], sems[p]).wait()
                _one_b(b, bufs[p], g * _RING + p)
                nb = jnp.minimum(b + _RING, _CB - 1)
                _rows_copy(nb, bufs[p], sems[p]).start()
            return carry
        lax.fori_loop(0, _OUTROWS // _RING, _g, 0)
        pltpu.sync_copy(out_v, ws_out.at[pl.ds(base + half * _OUTROWS, _OUTROWS)])

    for p in range(_RING):
        _rows_copy(_CB - 1, bufs[p], sems[p]).wait()

    for j in range(_NJ):
        cp = pltpu.make_async_copy(table_hbm.at[nid_v.at[j]], self_v, sem_self)
        cp.start()
        cp.wait()
        pltpu.sync_copy(self_v, xs_out.at[pl.ds(base + j * _IDXW, _IDXW)])


_sc_gather = functools.partial(
    pl.kernel,
    out_type=(jax.ShapeDtypeStruct((B, D), jnp.float32),
              jax.ShapeDtypeStruct((B, D), jnp.float32)),
    mesh=plsc.VectorSubcoreMesh(core_axis_name="c", subcore_axis_name="s"),
    scratch_types=[
        pltpu.VMEM((_NJ, _IDXW), jnp.int32),
        pltpu.VMEM((_CB, K), jnp.int32),
        pltpu.VMEM((_CB, K), jnp.float32),
        pltpu.VMEM((_IDXW, D), jnp.float32),
        pltpu.VMEM((K, D), jnp.float32),
        pltpu.VMEM((K, D), jnp.float32),
        pltpu.VMEM((K, D), jnp.float32),
        pltpu.VMEM((K, D), jnp.float32),
        pltpu.VMEM((_OUTROWS, D), jnp.float32),
        pltpu.SemaphoreType.DMA,
        pltpu.SemaphoreType.DMA,
        pltpu.SemaphoreType.DMA,
        pltpu.SemaphoreType.DMA,
        pltpu.SemaphoreType.DMA,
        pltpu.SemaphoreType.DMA,
        pltpu.SemaphoreType.DMA,
    ],
    compiler_params=pltpu.CompilerParams(use_tc_tiling_on_sc=False),
)(_sc_body)


def _tc_body(w_ref, x_ref, wn_ref, ws_ref, wc_ref, b1_ref, bc_ref, o_ref):
    h = jnp.dot(w_ref[...], wn_ref[...], preferred_element_type=jnp.float32)
    h = h + jnp.dot(x_ref[...], ws_ref[...], preferred_element_type=jnp.float32)
    h = h + b1_ref[...]
    o = jnp.dot(h, wc_ref[...], preferred_element_type=jnp.float32) + bc_ref[...]
    o_ref[...] = jnp.maximum(o, 0.0)


_TC_BLK = 2048


def _tc_linear(wsum, xs, wnT, wsT, wcT, b1, bc):
    return pl.pallas_call(
        _tc_body,
        grid=(B // _TC_BLK,),
        in_specs=[
            pl.BlockSpec((_TC_BLK, D), lambda i: (i, 0)),
            pl.BlockSpec((_TC_BLK, D), lambda i: (i, 0)),
            pl.BlockSpec((D, D), lambda i: (0, 0)),
            pl.BlockSpec((D, D), lambda i: (0, 0)),
            pl.BlockSpec((D, D), lambda i: (0, 0)),
            pl.BlockSpec((1, D), lambda i: (0, 0)),
            pl.BlockSpec((1, D), lambda i: (0, 0)),
        ],
        out_specs=pl.BlockSpec((_TC_BLK, D), lambda i: (i, 0)),
        out_shape=jax.ShapeDtypeStruct((B, D), jnp.float32),
    )(wsum, xs, wnT, wsT, wcT, b1, bc)


def kernel(node_ids, global_emb_table, offline_nbr_ids, offline_nbr_weights,
           Wn, bn, Ws, bs, Wc, bc):
    nid2 = node_ids.astype(jnp.int32).reshape(B // _IDXW, _IDXW)
    xs, ws = _sc_gather(nid2, global_emb_table,
                        offline_nbr_ids.astype(jnp.int32), offline_nbr_weights)
    l = Wn.shape[0] - 1
    b1 = (bn[l] + bs[l]).reshape(1, D)
    return _tc_linear(ws, xs, Wn[l].T, Ws[l].T, Wc[l].T, b1, bc[l].reshape(1, D))


# speedup vs baseline: 1.2102x; 1.0140x over previous
import functools

import jax
import jax.numpy as jnp
from jax import lax
from jax.experimental import pallas as pl
from jax.experimental.pallas import tpu as pltpu
from jax.experimental.pallas import tpu_sc as plsc

N = 50000
K = 32
D = 128
B = 16384

_info = plsc.get_sparse_core_info()
_NC, _NS, _LANES = _info.num_cores, _info.num_subcores, _info.num_lanes
_NW = _NC * _NS
_CB = B // _NW
_RING = 4
_OUTROWS = 128
_IDXW = 128
_NJ = _CB // _IDXW


def _sc_body(nid_hbm, table_hbm, nbrids_hbm, nbrw_hbm, xs_out, ws_out,
             nid_v, nbrids_v, nbrw_v, self_v, rows0, rows1, rows2, rows3,
             out_v, sem_ids, sem_w, sem_self, sem0, sem1, sem2, sem3):
    wid = lax.axis_index("s") * _NC + lax.axis_index("c")
    base = wid * _CB

    pltpu.sync_copy(nid_hbm.at[pl.ds(wid * _NJ, _NJ)], nid_v)

    cps = [pltpu.make_async_copy(
        nbrids_hbm.at[nid_v.at[j]], nbrids_v.at[pl.ds(j * _IDXW, _IDXW)],
        sem_ids) for j in range(_NJ)]
    cps += [pltpu.make_async_copy(
        nbrw_hbm.at[nid_v.at[j]], nbrw_v.at[pl.ds(j * _IDXW, _IDXW)],
        sem_w) for j in range(_NJ)]
    for cp in cps:
        cp.start()

    for j in range(_NJ):
        cp = pltpu.make_async_copy(table_hbm.at[nid_v.at[j]], self_v, sem_self)
        cp.start()
        cp.wait()
        pltpu.sync_copy(self_v, xs_out.at[pl.ds(base + j * _IDXW, _IDXW)])

    for cp in cps:
        cp.wait()

    bufs = (rows0, rows1, rows2, rows3)
    sems = (sem0, sem1, sem2, sem3)

    def _rows_copy(b, buf, sem):
        return pltpu.make_async_copy(table_hbm.at[nbrids_v.at[b]], buf, sem)

    for p in range(_RING):
        _rows_copy(p, bufs[p], sems[p]).start()

    def _one_b(b, buf, row_local):
        w0 = nbrw_v[b, pl.ds(0, 16)]
        w1 = nbrw_v[b, pl.ds(16, 16)]
        accs = [jnp.zeros((16,), jnp.float32) for _ in range(D // 16)]
        for k in range(K):
            wsc = (w0 if k < 16 else w1)[k % 16]
            for fb in range(D // 16):
                accs[fb] = accs[fb] + buf[k, pl.ds(fb * 16, 16)] * wsc
        for fb in range(D // 16):
            out_v[row_local, pl.ds(fb * 16, 16)] = accs[fb]

    for half in range(_CB // _OUTROWS):
        def _g(g, carry):
            for p in range(_RING):
                b = half * _OUTROWS + g * _RING + p
                _rows_copy(b, bufs[p], sems[p]).wait()
                _one_b(b, bufs[p], g * _RING + p)
                nb = jnp.minimum(b + _RING, _CB - 1)
                _rows_copy(nb, bufs[p], sems[p]).start()
            return carry
        lax.fori_loop(0, _OUTROWS // _RING, _g, 0)
        pltpu.sync_copy(out_v, ws_out.at[pl.ds(base + half * _OUTROWS, _OUTROWS)])

    for p in range(_RING):
        _rows_copy(_CB - 1, bufs[p], sems[p]).wait()


_sc_gather = functools.partial(
    pl.kernel,
    out_type=(jax.ShapeDtypeStruct((B, D), jnp.float32),
              jax.ShapeDtypeStruct((B, D), jnp.float32)),
    mesh=plsc.VectorSubcoreMesh(core_axis_name="c", subcore_axis_name="s"),
    scratch_types=[
        pltpu.VMEM((_NJ, _IDXW), jnp.int32),
        pltpu.VMEM((_CB, K), jnp.int32),
        pltpu.VMEM((_CB, K), jnp.float32),
        pltpu.VMEM((_IDXW, D), jnp.float32),
        pltpu.VMEM((K, D), jnp.float32),
        pltpu.VMEM((K, D), jnp.float32),
        pltpu.VMEM((K, D), jnp.float32),
        pltpu.VMEM((K, D), jnp.float32),
        pltpu.VMEM((_OUTROWS, D), jnp.float32),
        pltpu.SemaphoreType.DMA,
        pltpu.SemaphoreType.DMA,
        pltpu.SemaphoreType.DMA,
        pltpu.SemaphoreType.DMA,
        pltpu.SemaphoreType.DMA,
        pltpu.SemaphoreType.DMA,
        pltpu.SemaphoreType.DMA,
    ],
    compiler_params=pltpu.CompilerParams(use_tc_tiling_on_sc=False),
)(_sc_body)


def _tc_body(w_ref, x_ref, wn_ref, ws_ref, wc_ref, b1_ref, bc_ref, o_ref):
    h = jnp.dot(w_ref[...], wn_ref[...], preferred_element_type=jnp.float32)
    h = h + jnp.dot(x_ref[...], ws_ref[...], preferred_element_type=jnp.float32)
    h = h + b1_ref[...]
    o = jnp.dot(h, wc_ref[...], preferred_element_type=jnp.float32) + bc_ref[...]
    o_ref[...] = jnp.maximum(o, 0.0)


_TC_BLK = 4096


def _tc_linear(wsum, xs, wnT, wsT, wcT, b1, bc):
    return pl.pallas_call(
        _tc_body,
        grid=(B // _TC_BLK,),
        in_specs=[
            pl.BlockSpec((_TC_BLK, D), lambda i: (i, 0)),
            pl.BlockSpec((_TC_BLK, D), lambda i: (i, 0)),
            pl.BlockSpec((D, D), lambda i: (0, 0)),
            pl.BlockSpec((D, D), lambda i: (0, 0)),
            pl.BlockSpec((D, D), lambda i: (0, 0)),
            pl.BlockSpec((1, D), lambda i: (0, 0)),
            pl.BlockSpec((1, D), lambda i: (0, 0)),
        ],
        out_specs=pl.BlockSpec((_TC_BLK, D), lambda i: (i, 0)),
        out_shape=jax.ShapeDtypeStruct((B, D), jnp.float32),
    )(wsum, xs, wnT, wsT, wcT, b1, bc)


def kernel(node_ids, global_emb_table, offline_nbr_ids, offline_nbr_weights,
           Wn, bn, Ws, bs, Wc, bc):
    nid2 = node_ids.astype(jnp.int32).reshape(B // _IDXW, _IDXW)
    xs, ws = _sc_gather(nid2, global_emb_table,
                        offline_nbr_ids.astype(jnp.int32), offline_nbr_weights)
    l = Wn.shape[0] - 1
    b1 = (bn[l] + bs[l]).reshape(1, D)
    return _tc_linear(ws, xs, Wn[l].T, Ws[l].T, Wc[l].T, b1, bc[l].reshape(1, D))


# speedup vs baseline: 1.2296x; 1.0160x over previous
import functools

import jax
import jax.numpy as jnp
from jax import lax
from jax.experimental import pallas as pl
from jax.experimental.pallas import tpu as pltpu
from jax.experimental.pallas import tpu_sc as plsc

N = 50000
K = 32
D = 128
B = 16384

_info = plsc.get_sparse_core_info()
_NC, _NS, _LANES = _info.num_cores, _info.num_subcores, _info.num_lanes
_NW = _NC * _NS
_CB = B // _NW
_RING = 4
_OUTROWS = 256
_IDXW = 128
_NJ = _CB // _IDXW


def _sc_body(nid_hbm, table_hbm, nbrids_hbm, nbrw_hbm, xs_out, ws_out,
             nid_v, nbrids_v, nbrw_v, self_v, rows0, rows1, rows2, rows3,
             out_v, sem_ids, sem_w, sem_self, sem0, sem1, sem2, sem3):
    wid = lax.axis_index("s") * _NC + lax.axis_index("c")
    base = wid * _CB

    pltpu.sync_copy(nid_hbm.at[pl.ds(wid * _NJ, _NJ)], nid_v)

    cps = [pltpu.make_async_copy(
        nbrids_hbm.at[nid_v.at[j]], nbrids_v.at[pl.ds(j * _IDXW, _IDXW)],
        sem_ids) for j in range(_NJ)]
    cps += [pltpu.make_async_copy(
        nbrw_hbm.at[nid_v.at[j]], nbrw_v.at[pl.ds(j * _IDXW, _IDXW)],
        sem_w) for j in range(_NJ)]
    for cp in cps:
        cp.start()

    for j in range(_NJ):
        cp = pltpu.make_async_copy(table_hbm.at[nid_v.at[j]], self_v, sem_self)
        cp.start()
        cp.wait()
        pltpu.sync_copy(self_v, xs_out.at[pl.ds(base + j * _IDXW, _IDXW)])

    for cp in cps:
        cp.wait()

    bufs = (rows0, rows1, rows2, rows3)
    sems = (sem0, sem1, sem2, sem3)

    def _rows_copy(b, buf, sem):
        return pltpu.make_async_copy(table_hbm.at[nbrids_v.at[b]], buf, sem)

    for p in range(_RING):
        _rows_copy(p, bufs[p], sems[p]).start()

    def _one_b(b, buf, row_local):
        w0 = nbrw_v[b, pl.ds(0, 16)]
        w1 = nbrw_v[b, pl.ds(16, 16)]
        accs = [jnp.zeros((16,), jnp.float32) for _ in range(D // 16)]
        for k in range(K):
            wsc = (w0 if k < 16 else w1)[k % 16]
            for fb in range(D // 16):
                accs[fb] = accs[fb] + buf[k, pl.ds(fb * 16, 16)] * wsc
        for fb in range(D // 16):
            out_v[row_local, pl.ds(fb * 16, 16)] = accs[fb]

    for half in range(_CB // _OUTROWS):
        def _g(g, carry):
            for p in range(_RING):
                b = half * _OUTROWS + g * _RING + p
                _rows_copy(b, bufs[p], sems[p]).wait()
                _one_b(b, bufs[p], g * _RING + p)
                nb = jnp.minimum(b + _RING, _CB - 1)
                _rows_copy(nb, bufs[p], sems[p]).start()
            return carry
        lax.fori_loop(0, _OUTROWS // _RING, _g, 0)
        pltpu.sync_copy(out_v, ws_out.at[pl.ds(base + half * _OUTROWS, _OUTROWS)])

    for p in range(_RING):
        _rows_copy(_CB - 1, bufs[p], sems[p]).wait()


_sc_gather = functools.partial(
    pl.kernel,
    out_type=(jax.ShapeDtypeStruct((B, D), jnp.float32),
              jax.ShapeDtypeStruct((B, D), jnp.float32)),
    mesh=plsc.VectorSubcoreMesh(core_axis_name="c", subcore_axis_name="s"),
    scratch_types=[
        pltpu.VMEM((_NJ, _IDXW), jnp.int32),
        pltpu.VMEM((_CB, K), jnp.int32),
        pltpu.VMEM((_CB, K), jnp.float32),
        pltpu.VMEM((_IDXW, D), jnp.float32),
        pltpu.VMEM((K, D), jnp.float32),
        pltpu.VMEM((K, D), jnp.float32),
        pltpu.VMEM((K, D), jnp.float32),
        pltpu.VMEM((K, D), jnp.float32),
        pltpu.VMEM((_OUTROWS, D), jnp.float32),
        pltpu.SemaphoreType.DMA,
        pltpu.SemaphoreType.DMA,
        pltpu.SemaphoreType.DMA,
        pltpu.SemaphoreType.DMA,
        pltpu.SemaphoreType.DMA,
        pltpu.SemaphoreType.DMA,
        pltpu.SemaphoreType.DMA,
    ],
    compiler_params=pltpu.CompilerParams(use_tc_tiling_on_sc=False),
)(_sc_body)


def _tc_body(w_ref, x_ref, wn_ref, ws_ref, wc_ref, b1_ref, bc_ref, o_ref):
    h = jnp.dot(w_ref[...], wn_ref[...], preferred_element_type=jnp.float32)
    h = h + jnp.dot(x_ref[...], ws_ref[...], preferred_element_type=jnp.float32)
    h = h + b1_ref[...]
    o = jnp.dot(h, wc_ref[...], preferred_element_type=jnp.float32) + bc_ref[...]
    o_ref[...] = jnp.maximum(o, 0.0)


_TC_BLK = 4096


def _tc_linear(wsum, xs, wnT, wsT, wcT, b1, bc):
    return pl.pallas_call(
        _tc_body,
        grid=(B // _TC_BLK,),
        in_specs=[
            pl.BlockSpec((_TC_BLK, D), lambda i: (i, 0)),
            pl.BlockSpec((_TC_BLK, D), lambda i: (i, 0)),
            pl.BlockSpec((D, D), lambda i: (0, 0)),
            pl.BlockSpec((D, D), lambda i: (0, 0)),
            pl.BlockSpec((D, D), lambda i: (0, 0)),
            pl.BlockSpec((1, D), lambda i: (0, 0)),
            pl.BlockSpec((1, D), lambda i: (0, 0)),
        ],
        out_specs=pl.BlockSpec((_TC_BLK, D), lambda i: (i, 0)),
        out_shape=jax.ShapeDtypeStruct((B, D), jnp.float32),
    )(wsum, xs, wnT, wsT, wcT, b1, bc)


def kernel(node_ids, global_emb_table, offline_nbr_ids, offline_nbr_weights,
           Wn, bn, Ws, bs, Wc, bc):
    nid2 = node_ids.astype(jnp.int32).reshape(B // _IDXW, _IDXW)
    xs, ws = _sc_gather(nid2, global_emb_table,
                        offline_nbr_ids.astype(jnp.int32), offline_nbr_weights)
    l = Wn.shape[0] - 1
    b1 = (bn[l] + bs[l]).reshape(1, D)
    return _tc_linear(ws, xs, Wn[l].T, Ws[l].T, Wc[l].T, b1, bc[l].reshape(1, D))
